# baseline (speedup 1.0000x reference)
import functools

import jax
import jax.numpy as jnp
from jax import lax
from jax.experimental import pallas as pl
from jax.experimental.pallas import tpu as pltpu
from jax.experimental.pallas import tpu_sc as plsc

BATCH = 16384
PEPTIDE = 31
VOCAB = 27
EMB = 64

NUM_CORES = 2
NUM_SUBCORES = 16
NUM_WORKERS = NUM_CORES * NUM_SUBCORES
LANES = 16

TROWS = VOCAB * PEPTIDE
TOKENS = BATCH * PEPTIDE
TOK_PER_W = TOKENS // NUM_WORKERS
ROWS_PER_W = BATCH // NUM_WORKERS
GROUP = 4
NGROUPS = ROWS_PER_W // GROUP
NBUF = 4
IDXW = PEPTIDE
EVECS = EMB // LANES


def _make_sc_kernel():
  mesh = plsc.VectorSubcoreMesh(core_axis_name="c", subcore_axis_name="s")

  return functools.partial(
      pl.kernel,
      mesh=mesh,
      out_type=jax.ShapeDtypeStruct((BATCH, PEPTIDE, EMB), jnp.float32),
      compiler_params=pltpu.CompilerParams(use_tc_tiling_on_sc=False),
      scratch_types=[
          pltpu.HBM((NUM_WORKERS * TROWS, EMB), jnp.float32),
          pltpu.VMEM((TROWS, EMB), jnp.float32),
          pltpu.VMEM((VOCAB, EMB), jnp.float32),
          pltpu.VMEM((PEPTIDE, EMB), jnp.float32),
          pltpu.VMEM((TOK_PER_W + LANES,), jnp.int32),
          pltpu.VMEM((ROWS_PER_W, IDXW), jnp.int32),
      ]
      + [pltpu.VMEM((GROUP, PEPTIDE, EMB), jnp.float32) for _ in range(NBUF)]
      + [pltpu.SemaphoreType.DMA for _ in range(2 * NBUF)],
  )


def _sc_body(aa_hbm, pos_hbm, x_hbm, out_hbm,
             ctab_hbm, tabv, aav, posv, xv, idxv,
             buf0, buf1, buf2, buf3,
             gsem0, gsem1, gsem2, gsem3,
             wsem0, wsem1, wsem2, wsem3):
  bufs = (buf0, buf1, buf2, buf3)
  gsems = (gsem0, gsem1, gsem2, gsem3)
  wsems = (wsem0, wsem1, wsem2, wsem3)

  wid = lax.axis_index("s") * NUM_CORES + lax.axis_index("c")
  row0 = wid * ROWS_PER_W
  tbase = wid * TROWS

  pltpu.sync_copy(x_hbm.at[wid], xv.at[pl.ds(0, TOK_PER_W)])
  pltpu.sync_copy(aa_hbm, aav)
  pltpu.sync_copy(pos_hbm, posv)

  def build_vocab(v, carry):
    avecs = [aav[v, pl.ds(e * LANES, LANES)] for e in range(EVECS)]

    def build_pos(p, carry2):
      for e in range(EVECS):
        tabv[v * PEPTIDE + p, pl.ds(e * LANES, LANES)] = (
            avecs[e] + posv[p, pl.ds(e * LANES, LANES)])
      return carry2

    lax.fori_loop(0, PEPTIDE, build_pos, 0)
    return carry

  lax.fori_loop(0, VOCAB, build_vocab, 0)

  pltpu.sync_copy(tabv, ctab_hbm.at[pl.ds(tbase, TROWS)])

  iota = lax.iota(jnp.int32, LANES)
  base_lo = iota + tbase
  base_hi = iota + (15 + tbase)

  def build_idx(g, carry):
    x_lo = xv[pl.ds(g * PEPTIDE, LANES)]
    x_hi = xv[pl.ds(g * PEPTIDE + 15, LANES)]
    idxv[g, pl.ds(0, LANES)] = x_lo * PEPTIDE + base_lo
    idxv[g, pl.ds(15, LANES)] = x_hi * PEPTIDE + base_hi
    return carry

  lax.fori_loop(0, ROWS_PER_W, build_idx, 0)

  def start_gathers(gg, b):
    for bb in range(GROUP):
      pltpu.async_copy(
          ctab_hbm.at[idxv.at[gg * GROUP + bb]],
          bufs[b].at[bb], gsems[b])

  def wait_gathers(gg, b):
    for bb in range(GROUP):
      pltpu.make_async_copy(
          ctab_hbm.at[idxv.at[gg * GROUP + bb]],
          bufs[b].at[bb], gsems[b]).wait()

  def start_write(gg, b):
    pltpu.async_copy(bufs[b], out_hbm.at[pl.ds(row0 + gg * GROUP, GROUP)],
                     wsems[b])

  def drain_write(b):
    pltpu.make_async_copy(bufs[b], out_hbm.at[pl.ds(row0, GROUP)],
                          wsems[b]).wait()

  start_gathers(0, 0)
  start_gathers(1, 1)

  def group_step(i, carry):
    for b in range(NBUF):
      gg = i * NBUF + b
      bn = (b + 2) % NBUF

      @pl.when(jnp.logical_and(gg >= 2, gg + 2 < NGROUPS))
      def _():
        drain_write(bn)

      @pl.when(gg + 2 < NGROUPS)
      def _():
        start_gathers(gg + 2, bn)

      wait_gathers(gg, b)
      start_write(gg, b)
    return carry

  lax.fori_loop(0, NGROUPS // NBUF, group_step, 0)

  for b in range(NBUF):
    drain_write(b)


def kernel(x, aa_table, pos_table):
  x2 = x.astype(jnp.int32).reshape(NUM_WORKERS, TOK_PER_W)
  return _make_sc_kernel()(_sc_body)(aa_table, pos_table, x2)
